# Initial kernel scaffold; baseline (speedup 1.0000x reference)
import jax
import jax.numpy as jnp
from jax.experimental import pallas as pl


def kernel(x_operator, x_table, x_column, x_predicate, ei_scannedby, ei_filters, ei_outputby, ei_connects, ei_calledby, ei_table_selfloop, ei_column_selfloop, batch_operator, params):
    raise NotImplementedError("write your pallas kernel here")


# speedup vs baseline: 15.3848x; 15.3848x over previous
import functools

import jax
import jax.numpy as jnp
from jax import lax
from jax.experimental import pallas as pl
from jax.experimental.pallas import tpu as pltpu
from jax.experimental.pallas import tpu_sc as plsc

NTYPES = ['operator', 'table', 'column', 'predicate']
ETYPES = [('table', 'scannedby', 'operator'),
          ('predicate', 'filters', 'operator'),
          ('column', 'outputby', 'operator'),
          ('column', 'connects', 'predicate'),
          ('operator', 'calledby', 'operator'),
          ('table', 'selfloop', 'table'),
          ('column', 'selfloop', 'column')]
NNODES = {'operator': 50000, 'table': 10000, 'column': 50000, 'predicate': 50000}
HID = 64
HALF = 32
K = 2
LANES = 16
BATCH = 512


def _ceil_to(x, m):
    return (x + m - 1) // m * m


def _lk(t):
    return jnp.where(t >= 0.0, t, 0.2 * t)


@functools.cache
def _edge_kernel(E_pad, n_src, n_dst):
    R = _ceil_to(n_dst + 1, 10240)
    NSUB = E_pad // 128 // 16
    G = NSUB // K
    R16 = R // 16
    mesh = plsc.VectorSubcoreMesh(core_axis_name="c", subcore_axis_name="s")

    @functools.partial(
        pl.kernel,
        mesh=mesh,
        compiler_params=pltpu.CompilerParams(needs_layout_passes=False, use_tc_tiling_on_sc=False),
        out_type=[
            jax.ShapeDtypeStruct((R, HALF), jnp.float32),
            jax.ShapeDtypeStruct((R, HALF), jnp.float32),
            jax.ShapeDtypeStruct((R,), jnp.float32),
        ],
        scratch_types=[
            pltpu.VMEM_SHARED((R, HALF), jnp.float32),
            pltpu.VMEM_SHARED((R,), jnp.float32),
            pltpu.VMEM_SHARED((n_src,), jnp.float32),
            pltpu.VMEM_SHARED((n_dst,), jnp.float32),
            pltpu.VMEM((16,), jnp.float32),
            pltpu.VMEM((K, 128), jnp.int32),
            pltpu.VMEM((K, 128), jnp.int32),
            pltpu.VMEM((K * 128, HALF), jnp.float32),
            pltpu.VMEM((K, 128), jnp.float32),
            pltpu.VMEM((K, 128), jnp.float32),
            pltpu.VMEM((K, 128), jnp.float32),
            pltpu.VMEM((640,), jnp.float32),
            pltpu.SemaphoreType.DMA,
            pltpu.SemaphoreType.DMA,
            pltpu.SemaphoreType.DMA,
        ],
    )
    def kfn(ei0_h, ei1_h, asrc_h, adst_h, m8_h, xlo_h, xhi_h,
            acc0_h, acc1_h, den_h,
            acc_sh, den_sh, asrc_sh, adst_sh, m_v,
            idx0_v, idx1_v, rows_v, e_v, sv_v, dv_v, zden_v,
            gsem, asem, ssem):
        c = lax.axis_index("c")
        t = lax.axis_index("s")
        z16 = jnp.zeros((LANES,), jnp.float32)

        def zfill(i, _):
            rows_v[i, pl.ds(0, 16)] = z16
            rows_v[i, pl.ds(16, 16)] = z16
            return 0
        lax.fori_loop(0, 128, zfill, 0)

        def zfill1(i, _):
            zden_v[pl.ds(i * 16, 16)] = z16
            return 0
        lax.fori_loop(0, 40, zfill1, 0)

        def zacc(i, _):
            pltpu.sync_copy(rows_v.at[pl.ds(0, 128)],
                            acc_sh.at[pl.ds(t * R16 + i * 128, 128)])
            return 0
        lax.fori_loop(0, R16 // 128, zacc, 0)

        def zden(i, _):
            pltpu.sync_copy(zden_v, den_sh.at[pl.ds(t * R16 + i * 640, 640)])
            return 0
        lax.fori_loop(0, R16 // 640, zden, 0)

        @pl.when(t == 0)
        def _():
            pltpu.sync_copy(asrc_h, asrc_sh)
            pltpu.sync_copy(adst_h, adst_sh)
        pltpu.sync_copy(m8_h, m_v)
        plsc.subcore_barrier()
        m = m_v[pl.ds(0, 16)]

        def group(g, _):
            row0 = t * NSUB + g * K

            @pl.when(g > 0)
            def _():
                for k in range(K):
                    pltpu.make_async_copy(
                        xlo_h.at[pl.ds(0, 128)],
                        rows_v.at[pl.ds(k * 128, 128)], ssem).wait()
                    pltpu.make_async_copy(
                        den_h.at[pl.ds(0, 128)], e_v.at[k], ssem).wait()

            pltpu.sync_copy(ei0_h.at[pl.ds(row0, K)], idx0_v)
            pltpu.sync_copy(ei1_h.at[pl.ds(row0, K)], idx1_v)

            @pl.when(c == 0)
            def _():
                for k in range(K):
                    pltpu.async_copy(xlo_h.at[idx0_v.at[k]],
                                     rows_v.at[pl.ds(k * 128, 128)], gsem)

            @pl.when(c == 1)
            def _():
                for k in range(K):
                    pltpu.async_copy(xhi_h.at[idx0_v.at[k]],
                                     rows_v.at[pl.ds(k * 128, 128)], gsem)

            for k in range(K):
                pltpu.async_copy(asrc_sh.at[idx0_v.at[k]], sv_v.at[k], asem)
                pltpu.async_copy(adst_sh.at[idx1_v.at[k]], dv_v.at[k], asem)
            for k in range(K):
                pltpu.make_async_copy(den_h.at[pl.ds(0, 128)], sv_v.at[k],
                                      asem).wait()
                pltpu.make_async_copy(den_h.at[pl.ds(0, 128)], dv_v.at[k],
                                      asem).wait()

            for k in range(K):
                for l in range(8):
                    sv = sv_v[k, pl.ds(l * 16, 16)]
                    dv = dv_v[k, pl.ds(l * 16, 16)]
                    a = _lk(sv + dv)
                    bb = _lk(dv + m)
                    e_v[k, pl.ds(l * 16, 16)] = jnp.exp(a - bb)

            for k in range(K):
                pltpu.make_async_copy(
                    xlo_h.at[pl.ds(0, 128)],
                    rows_v.at[pl.ds(k * 128, 128)], gsem).wait()

            for k in range(K):
                for l in range(8):
                    ev16 = e_v[k, pl.ds(l * 16, 16)]
                    for ii in range(16):
                        r = k * 128 + l * 16 + ii
                        ev = ev16[ii]
                        rows_v[r, pl.ds(0, 16)] = rows_v[r, pl.ds(0, 16)] * ev
                        rows_v[r, pl.ds(16, 16)] = rows_v[r, pl.ds(16, 16)] * ev

            for k in range(K):
                pltpu.async_copy(rows_v.at[pl.ds(k * 128, 128)],
                                 acc_sh.at[idx1_v.at[k]], ssem, add=True)
                pltpu.async_copy(e_v.at[k], den_sh.at[idx1_v.at[k]],
                                 ssem, add=True)
            return 0
        lax.fori_loop(0, G, group, 0)

        for k in range(K):
            pltpu.make_async_copy(xlo_h.at[pl.ds(0, 128)],
                                  rows_v.at[pl.ds(k * 128, 128)], ssem).wait()
            pltpu.make_async_copy(den_h.at[pl.ds(0, 128)], e_v.at[k],
                                  ssem).wait()
        plsc.subcore_barrier()

        @pl.when(c == 0)
        def _():
            pltpu.sync_copy(acc_sh.at[pl.ds(t * R16, R16)],
                            acc0_h.at[pl.ds(t * R16, R16)])
            pltpu.sync_copy(den_sh.at[pl.ds(t * R16, R16)],
                            den_h.at[pl.ds(t * R16, R16)])

        @pl.when(c == 1)
        def _():
            pltpu.sync_copy(acc_sh.at[pl.ds(t * R16, R16)],
                            acc1_h.at[pl.ds(t * R16, R16)])
    return kfn


def _edge_pass(xn_src, a_src, a_dst, ei, n_src, n_dst):
    E = ei.shape[1]
    E_pad = _ceil_to(E, 16 * 128 * K)
    ei0 = jnp.concatenate([ei[0], jnp.zeros((E_pad - E,), jnp.int32)]).reshape(-1, 128)
    ei1 = jnp.concatenate([ei[1], jnp.full((E_pad - E,), n_dst, jnp.int32)]).reshape(-1, 128)
    m8 = jnp.full((16,), jnp.max(a_src), jnp.float32)
    xlo = xn_src[:, :HALF]
    xhi = xn_src[:, HALF:]
    acc0, acc1, den = _edge_kernel(E_pad, n_src, n_dst)(
        ei0, ei1, a_src, a_dst, m8, xlo, xhi)
    acc = jnp.concatenate([acc0[:n_dst], acc1[:n_dst]], axis=1)
    den = jnp.maximum(den[:n_dst], 1e-30)
    return jax.nn.relu(acc / den[:, None])


@functools.cache
def _pool_kernel(Np):
    RB = 1024
    NSUB = Np // 128 // 32
    RB16 = RB // 16
    mesh = plsc.VectorSubcoreMesh(core_axis_name="c", subcore_axis_name="s")

    @functools.partial(
        pl.kernel,
        mesh=mesh,
        compiler_params=pltpu.CompilerParams(needs_layout_passes=False, use_tc_tiling_on_sc=False),
        out_type=[
            jax.ShapeDtypeStruct((2, RB, HID), jnp.float32),
            jax.ShapeDtypeStruct((2, RB), jnp.float32),
        ],
        scratch_types=[
            pltpu.VMEM_SHARED((RB, HID), jnp.float32),
            pltpu.VMEM_SHARED((RB,), jnp.float32),
            pltpu.VMEM((128, HID), jnp.float32),
            pltpu.VMEM((NSUB, 128), jnp.int32),
            pltpu.VMEM((128,), jnp.float32),
            pltpu.SemaphoreType.DMA,
    ],
    )
    def kfn(sv_h, bid_h, sums_h, cnt_h,
            sums_sh, cnt_sh, rows_v, bid_v, ones_v, sem):
        c = lax.axis_index("c")
        t = lax.axis_index("s")
        w = c * 16 + t
        z16 = jnp.zeros((LANES,), jnp.float32)
        o16 = jnp.ones((LANES,), jnp.float32)

        def zfill(i, _):
            for q in range(4):
                rows_v[i, pl.ds(q * 16, 16)] = z16
            return 0
        lax.fori_loop(0, 128, zfill, 0)
        for i in range(8):
            ones_v[pl.ds(i * 16, 16)] = o16

        pltpu.sync_copy(rows_v.at[pl.ds(0, 64)], sums_sh.at[pl.ds(t * 64, 64)])
        pltpu.sync_copy(rows_v.at[0, pl.ds(0, 64)], cnt_sh.at[pl.ds(t * 64, 64)])
        pltpu.sync_copy(bid_h.at[pl.ds(w * NSUB, NSUB)], bid_v)
        plsc.subcore_barrier()

        def grp(j, _):
            pltpu.sync_copy(sv_h.at[pl.ds((w * NSUB + j) * 128, 128)], rows_v)
            pltpu.sync_copy(rows_v, sums_sh.at[bid_v.at[j]], add=True)
            pltpu.sync_copy(ones_v, cnt_sh.at[bid_v.at[j]], add=True)
            return 0
        lax.fori_loop(0, NSUB, grp, 0)
        plsc.subcore_barrier()

        pltpu.sync_copy(sums_sh.at[pl.ds(t * 64, 64)],
                        sums_h.at[c].at[pl.ds(t * 64, 64)])
        pltpu.sync_copy(cnt_sh.at[pl.ds(t * 64, 64)],
                        cnt_h.at[c].at[pl.ds(t * 64, 64)])
    return kfn


def _layernorm(x, g, b):
    mu = jnp.mean(x, axis=-1, keepdims=True)
    var = jnp.mean((x - mu) ** 2, axis=-1, keepdims=True)
    return (x - mu) / jnp.sqrt(var + 1e-5) * g + b


def _semantic(stk, cp):
    score = jnp.sum(cp['q'] * jnp.mean(jnp.tanh(
        stk @ cp['k_lin_w'] + cp['k_lin_b']), axis=1), axis=-1)
    attn = jax.nn.softmax(score, axis=0)
    return jnp.sum(attn[:, None, None] * stk, axis=0)


def _conv(xn, eid, cp, needed_dst):
    outs = {nt: [] for nt in NTYPES}
    for et in ETYPES:
        src, _, dst = et
        if dst not in needed_dst:
            continue
        kk = '__'.join(et)
        a_src = jnp.sum(xn[src].reshape(-1, 1, HID) * cp['lin_src_' + kk],
                        axis=-1)[:, 0]
        a_dst = jnp.sum(xn[dst].reshape(-1, 1, HID) * cp['lin_dst_' + kk],
                        axis=-1)[:, 0]
        o = _edge_pass(xn[src], a_src, a_dst, eid[et],
                       NNODES[src], NNODES[dst])
        outs[dst].append(o)
    return {nt: _semantic(jnp.stack(outs[nt], 0), cp) for nt in needed_dst}


def kernel(x_operator, x_table, x_column, x_predicate, ei_scannedby,
           ei_filters, ei_outputby, ei_connects, ei_calledby,
           ei_table_selfloop, ei_column_selfloop, batch_operator, params):
    p = params
    eid = dict(zip(ETYPES, [ei_scannedby, ei_filters, ei_outputby,
                            ei_connects, ei_calledby, ei_table_selfloop,
                            ei_column_selfloop]))
    xin = {'operator': x_operator, 'table': x_table, 'column': x_column,
           'predicate': x_predicate}
    c1, c2 = p['conv1'], p['conv2']

    x_dict = {nt: xin[nt] @ p['lin_' + nt + '_w'] + p['lin_' + nt + '_b']
              for nt in NTYPES}
    xn1 = {nt: x_dict[nt] @ c1['proj_' + nt + '_w'] + c1['proj_' + nt + '_b']
           for nt in NTYPES}
    res1 = _conv(xn1, eid, c1, set(NTYPES))
    xmid = {nt: _layernorm(jax.nn.elu(res1[nt]), p['norm1_g'], p['norm1_b'])
            for nt in NTYPES}
    xn2 = {nt: xmid[nt] @ c2['proj_' + nt + '_w'] + c2['proj_' + nt + '_b']
           for nt in NTYPES}
    res2 = _conv(xn2, eid, c2, {'operator'})
    opf = _layernorm(jax.nn.elu(res2['operator']), p['norm2_g'], p['norm2_b'])

    n_op = opf.shape[0]
    Np = _ceil_to(n_op, 4096)
    sv = jnp.concatenate([opf, jnp.zeros((Np - n_op, HID), jnp.float32)], 0)
    bid = jnp.concatenate([batch_operator,
                           jnp.full((Np - n_op,), BATCH, jnp.int32)]).reshape(-1, 128)
    sums2, cnt2 = _pool_kernel(Np)(sv, bid)
    sums = (sums2[0] + sums2[1])[:BATCH]
    cnt = (cnt2[0] + cnt2[1])[:BATCH]
    pooled = sums / jnp.maximum(cnt, 1.0)[:, None]
    out = pooled @ p['lin_w'] + p['lin_b']
    return jnp.squeeze(out)
